# initial kernel scaffold (unmeasured)
import jax
import jax.numpy as jnp
from jax import lax
from jax.experimental import pallas as pl
from jax.experimental.pallas import tpu as pltpu

N_ROWS = 2048
N_COLS = 2048
N_DEV = 8
BLK = N_ROWS // N_DEV

MESH = pltpu.DeviceIdType.MESH


def kernel(partial, resid, gamma):
    p2 = partial.reshape(N_ROWS, N_COLS)
    g2 = gamma.reshape(1, N_COLS)

    def body(p_ref, r_ref, g_ref, out_ref,
             p_mine, p_send_f32, r_mine, send_buf, recv_buf,
             copy_sems, send_sems, recv_sems):
        my_x = lax.axis_index("x")
        my_y = lax.axis_index("y")
        my_z = lax.axis_index("z")

        b_me = 4 * my_y + 2 * my_x + my_z
        b_pr = 4 * (1 - my_y) + 2 * my_x + my_z

        barrier = pltpu.get_barrier_semaphore()
        for nbr in ((my_x, 1 - my_y, my_z),
                    (1 - my_x, my_y, my_z),
                    (my_x, my_y, 1 - my_z)):
            pltpu.semaphore_signal(barrier, inc=1, device_id=nbr,
                                   device_id_type=MESH)
        pltpu.semaphore_wait(barrier, 3)

        cp_send = pltpu.make_async_copy(
            p_ref.at[pl.ds(b_pr * BLK, BLK), :], p_send_f32, copy_sems.at[0])
        cp_mine = pltpu.make_async_copy(
            p_ref.at[pl.ds(b_me * BLK, BLK), :], p_mine, copy_sems.at[1])
        cp_res = pltpu.make_async_copy(
            r_ref.at[pl.ds(b_me * BLK, BLK), :], r_mine, copy_sems.at[2])
        cp_send.start()
        cp_mine.start()
        cp_res.start()

        cp_send.wait()
        send_buf[...] = p_send_f32[...].astype(jnp.bfloat16)
        rdma_y = pltpu.make_async_remote_copy(
            src_ref=send_buf, dst_ref=recv_buf,
            send_sem=send_sems.at[0], recv_sem=recv_sems.at[0],
            device_id=(my_x, 1 - my_y, my_z), device_id_type=MESH)
        rdma_y.start()
        cp_mine.wait()
        cp_res.wait()
        rdma_y.wait()

        y = p_mine[...] + recv_buf[...].astype(jnp.float32) + r_mine[...]
        ms = jnp.mean(y * y, axis=-1, keepdims=True)
        o = y * lax.rsqrt(ms + 1e-6) * g_ref[...]
        out_ref[pl.ds(b_me * BLK, BLK), :] = o.astype(jnp.bfloat16)

        starts = (b_me * BLK, (2 * my_y + my_x) * (2 * BLK), my_y * (4 * BLK))
        sizes = (BLK, 2 * BLK, 4 * BLK)
        partners = ((my_x, my_y, 1 - my_z),
                    (1 - my_x, my_y, my_z),
                    (my_x, 1 - my_y, my_z))
        for s in range(3):
            rdma = pltpu.make_async_remote_copy(
                src_ref=out_ref.at[pl.ds(starts[s], sizes[s]), :],
                dst_ref=out_ref.at[pl.ds(starts[s], sizes[s]), :],
                send_sem=send_sems.at[s + 1], recv_sem=recv_sems.at[s + 1],
                device_id=partners[s], device_id_type=MESH)
            rdma.start()
            rdma.wait()

    return pl.pallas_call(
        body,
        out_shape=jax.ShapeDtypeStruct((N_ROWS, N_COLS), jnp.bfloat16),
        in_specs=[pl.BlockSpec(memory_space=pltpu.ANY),
                  pl.BlockSpec(memory_space=pltpu.ANY),
                  pl.BlockSpec(memory_space=pltpu.VMEM)],
        out_specs=pl.BlockSpec(memory_space=pltpu.VMEM),
        scratch_shapes=[
            pltpu.VMEM((BLK, N_COLS), jnp.float32),
            pltpu.VMEM((BLK, N_COLS), jnp.float32),
            pltpu.VMEM((BLK, N_COLS), jnp.float32),
            pltpu.VMEM((BLK, N_COLS), jnp.bfloat16),
            pltpu.VMEM((BLK, N_COLS), jnp.bfloat16),
            pltpu.SemaphoreType.DMA((3,)),
            pltpu.SemaphoreType.DMA((4,)),
            pltpu.SemaphoreType.DMA((4,)),
        ],
        compiler_params=pltpu.CompilerParams(collective_id=0),
    )(p2, resid, g2)


# baseline (device time: 107105 ns/iter reference)
import jax
import jax.numpy as jnp
from jax import lax
from jax.experimental import pallas as pl
from jax.experimental.pallas import tpu as pltpu

N_ROWS = 2048
N_COLS = 2048
N_DEV = 8
BLK = N_ROWS // N_DEV

MESH = pltpu.DeviceIdType.MESH


def kernel(partial, resid, gamma):
    p2 = partial.reshape(N_ROWS, N_COLS)
    g2 = gamma.reshape(1, N_COLS)

    def body(p_ref, r_ref, g_ref, out_ref,
             p_mine, p_send_f32, r_mine, send_buf, recv_buf,
             copy_sems, send_sems, recv_sems):
        my_x = lax.axis_index("x")
        my_y = lax.axis_index("y")
        my_z = lax.axis_index("z")

        b_me = 4 * my_y + 2 * my_x + my_z
        b_pr = 4 * (1 - my_y) + 2 * my_x + my_z

        barrier = pltpu.get_barrier_semaphore()
        for nbr in ((my_x, 1 - my_y, my_z),
                    (1 - my_x, my_y, my_z),
                    (my_x, my_y, 1 - my_z)):
            pltpu.semaphore_signal(barrier, inc=1, device_id=nbr,
                                   device_id_type=MESH)
        pltpu.semaphore_wait(barrier, 3)

        cp_send = pltpu.make_async_copy(
            p_ref.at[pl.ds(b_pr * BLK, BLK), :], p_send_f32, copy_sems.at[0])
        cp_mine = pltpu.make_async_copy(
            p_ref.at[pl.ds(b_me * BLK, BLK), :], p_mine, copy_sems.at[1])
        cp_res = pltpu.make_async_copy(
            r_ref.at[pl.ds(b_me * BLK, BLK), :], r_mine, copy_sems.at[2])
        cp_send.start()
        cp_mine.start()
        cp_res.start()

        cp_send.wait()
        send_buf[...] = p_send_f32[...].astype(jnp.bfloat16)
        rdma_y = pltpu.make_async_remote_copy(
            src_ref=send_buf, dst_ref=recv_buf,
            send_sem=send_sems.at[0], recv_sem=recv_sems.at[0],
            device_id=(my_x, 1 - my_y, my_z), device_id_type=MESH)
        rdma_y.start()
        cp_mine.wait()
        cp_res.wait()
        rdma_y.wait()

        y = p_mine[...] + recv_buf[...].astype(jnp.float32) + r_mine[...]
        ms = jnp.mean(y * y, axis=-1, keepdims=True)
        o = y * lax.rsqrt(ms + 1e-6) * g_ref[...]
        out_ref[pl.ds(b_me * BLK, BLK), :] = o.astype(jnp.bfloat16)

        starts = (b_me * BLK, (2 * my_y + my_x) * (2 * BLK), my_y * (4 * BLK))
        sizes = (BLK, 2 * BLK, 4 * BLK)
        partners = ((my_x, my_y, 1 - my_z),
                    (1 - my_x, my_y, my_z),
                    (my_x, 1 - my_y, my_z))
        for s in range(3):
            rdma = pltpu.make_async_remote_copy(
                src_ref=out_ref.at[pl.ds(starts[s], sizes[s]), :],
                dst_ref=out_ref.at[pl.ds(starts[s], sizes[s]), :],
                send_sem=send_sems.at[s + 1], recv_sem=recv_sems.at[s + 1],
                device_id=partners[s], device_id_type=MESH)
            rdma.start()
            rdma.wait()

    return pl.pallas_call(
        body,
        out_shape=jax.ShapeDtypeStruct((N_ROWS, N_COLS), jnp.bfloat16),
        in_specs=[pl.BlockSpec(memory_space=pl.ANY),
                  pl.BlockSpec(memory_space=pl.ANY),
                  pl.BlockSpec(memory_space=pltpu.VMEM)],
        out_specs=pl.BlockSpec(memory_space=pltpu.VMEM),
        scratch_shapes=[
            pltpu.VMEM((BLK, N_COLS), jnp.float32),
            pltpu.VMEM((BLK, N_COLS), jnp.float32),
            pltpu.VMEM((BLK, N_COLS), jnp.float32),
            pltpu.VMEM((BLK, N_COLS), jnp.bfloat16),
            pltpu.VMEM((BLK, N_COLS), jnp.bfloat16),
            pltpu.SemaphoreType.DMA((3,)),
            pltpu.SemaphoreType.DMA((4,)),
            pltpu.SemaphoreType.DMA((4,)),
        ],
        compiler_params=pltpu.CompilerParams(collective_id=0),
    )(p2, resid, g2)
